# Initial kernel scaffold; baseline (speedup 1.0000x reference)
import jax
import jax.numpy as jnp
from jax.experimental import pallas as pl


def kernel(query_features, support_features):
    raise NotImplementedError("write your pallas kernel here")


# speedup vs baseline: 66.4595x; 66.4595x over previous
import jax
import jax.numpy as jnp
from jax.experimental import pallas as pl
from jax.experimental.pallas import tpu as pltpu

_K = 3.0


def _knn_step(q_ref, s_ref, out_ref):
    q = q_ref[0]
    s = s_ref[0, 0]

    qn = q * jax.lax.rsqrt(jnp.maximum(jnp.sum(q * q, axis=0, keepdims=True), 1e-24))
    sn = s * jax.lax.rsqrt(jnp.maximum(jnp.sum(s * s, axis=0, keepdims=True), 1e-24))

    sim = jax.lax.dot_general(
        qn.astype(jnp.bfloat16), sn.astype(jnp.bfloat16),
        dimension_numbers=(((0,), (0,)), ((), ())),
        preferred_element_type=jnp.float32,
    )

    m1 = jnp.max(sim, axis=1, keepdims=True)
    lt1 = sim < m1
    c1 = jnp.sum(jnp.where(lt1, 0.0, 1.0), axis=1, keepdims=True)
    x2 = jnp.where(lt1, sim, -3.0)
    m2 = jnp.max(x2, axis=1, keepdims=True)
    lt2 = x2 < m2
    c2 = jnp.sum(jnp.where(lt2, 0.0, 1.0), axis=1, keepdims=True)
    x3 = jnp.where(lt2, x2, -4.0)
    m3 = jnp.max(x3, axis=1, keepdims=True)

    a1 = jnp.minimum(c1, _K)
    a2 = jnp.minimum(_K - a1, c2)
    a3 = _K - a1 - a2
    top3 = m1 * a1 + m2 * a2 + m3 * a3

    out_ref[...] = jnp.full((1, 1, 8, 128), jnp.sum(top3), dtype=jnp.float32)


def kernel(query_features, support_features):
    B, D, h, w = query_features.shape
    Way = support_features.shape[1]
    hw = h * w
    q = query_features.reshape(B, D, hw)

    scores = pl.pallas_call(
        _knn_step,
        grid=(B, Way),
        in_specs=[
            pl.BlockSpec((1, D, hw), lambda b, c: (b, 0, 0)),
            pl.BlockSpec((1, 1, D, hw), lambda b, c: (b, c, 0, 0)),
        ],
        out_specs=pl.BlockSpec((1, 1, 8, 128), lambda b, c: (b, c, 0, 0)),
        out_shape=jax.ShapeDtypeStruct((B, Way, 8, 128), jnp.float32),
    )(q, support_features)
    return scores[:, :, 0, 0]


# speedup vs baseline: 117.5898x; 1.7693x over previous
import jax
import jax.numpy as jnp
from jax.experimental import pallas as pl

_K = 3.0
_SLAB = 16
_NEG = -1e9


def _knn_step(q_ref, s_ref, out_ref):
    q = q_ref[0]
    s = s_ref[0, 0]

    sn = s * jax.lax.rsqrt(jnp.maximum(jnp.sum(s * s, axis=0, keepdims=True), 1e-24))
    rq = jax.lax.rsqrt(jnp.maximum(jnp.sum(q * q, axis=0, keepdims=True), 1e-24))

    sim = jax.lax.dot_general(
        sn.astype(jnp.bfloat16), q.astype(jnp.bfloat16),
        dimension_numbers=(((0,), (0,)), ((), ())),
        preferred_element_type=jnp.float32,
    )

    n_s = sim.shape[0]
    sim3 = sim.reshape(n_s // _SLAB, _SLAB, sim.shape[1])
    a1 = jnp.full((_SLAB, sim.shape[1]), _NEG, dtype=jnp.float32)
    a2 = a1
    a3 = a1
    for i in range(sim3.shape[0]):
        v = sim3[i]
        t1 = jnp.maximum(a1, v)
        d1 = jnp.minimum(a1, v)
        t2 = jnp.maximum(a2, d1)
        d2 = jnp.minimum(a2, d1)
        t3 = jnp.maximum(a3, d2)
        a1, a2, a3 = t1, t2, t3

    cand = jnp.concatenate([a1, a2, a3], axis=0)
    m1 = jnp.max(cand, axis=0, keepdims=True)
    lt1 = cand < m1
    c1 = jnp.sum(jnp.where(lt1, 0.0, 1.0), axis=0, keepdims=True)
    x2 = jnp.where(lt1, cand, _NEG)
    m2 = jnp.max(x2, axis=0, keepdims=True)
    lt2 = x2 < m2
    c2 = jnp.sum(jnp.where(lt2, 0.0, 1.0), axis=0, keepdims=True)
    x3 = jnp.where(lt2, x2, _NEG)
    m3 = jnp.max(x3, axis=0, keepdims=True)

    b1 = jnp.minimum(c1, _K)
    b2 = jnp.minimum(_K - b1, c2)
    b3 = _K - b1 - b2
    top3 = m1 * b1 + m2 * b2 + m3 * b3

    out_ref[...] = jnp.full((1, 1, 8, 128), jnp.sum(top3 * rq), dtype=jnp.float32)


def kernel(query_features, support_features):
    B, D, h, w = query_features.shape
    Way = support_features.shape[1]
    hw = h * w
    q = query_features.reshape(B, D, hw)

    scores = pl.pallas_call(
        _knn_step,
        grid=(B, Way),
        in_specs=[
            pl.BlockSpec((1, D, hw), lambda b, c: (b, 0, 0)),
            pl.BlockSpec((1, 1, D, hw), lambda b, c: (b, c, 0, 0)),
        ],
        out_specs=pl.BlockSpec((1, 1, 8, 128), lambda b, c: (b, c, 0, 0)),
        out_shape=jax.ShapeDtypeStruct((B, Way, 8, 128), jnp.float32),
    )(q, support_features)
    return scores[:, :, 0, 0]


# speedup vs baseline: 187.4719x; 1.5943x over previous
import jax
import jax.numpy as jnp
from jax.experimental import pallas as pl

_K = 3.0
_SLAB = 16
_NEG = -1e9


def _way_score(q, s, rq):
    sn = s * jax.lax.rsqrt(jnp.maximum(jnp.sum(s * s, axis=0, keepdims=True), 1e-24))

    sim = jax.lax.dot_general(
        sn.astype(jnp.bfloat16), q.astype(jnp.bfloat16),
        dimension_numbers=(((0,), (0,)), ((), ())),
        preferred_element_type=jnp.float32,
    )

    n_s, n_q = sim.shape
    simb = sim.astype(jnp.bfloat16)
    sim3 = simb.reshape(n_s // _SLAB, _SLAB, n_q)
    a1 = jnp.full((_SLAB, n_q), _NEG, dtype=jnp.bfloat16)
    a2 = a1
    a3 = a1
    for i in range(sim3.shape[0]):
        v = sim3[i]
        t1 = jnp.maximum(a1, v)
        d1 = jnp.minimum(a1, v)
        t2 = jnp.maximum(a2, d1)
        d2 = jnp.minimum(a2, d1)
        t3 = jnp.maximum(a3, d2)
        a1, a2, a3 = t1, t2, t3

    cand = jnp.concatenate([a1, a2, a3], axis=0).astype(jnp.float32)
    m1 = jnp.max(cand, axis=0, keepdims=True)
    lt1 = cand < m1
    c1 = jnp.sum(jnp.where(lt1, 0.0, 1.0), axis=0, keepdims=True)
    x2 = jnp.where(lt1, cand, _NEG)
    m2 = jnp.max(x2, axis=0, keepdims=True)
    lt2 = x2 < m2
    c2 = jnp.sum(jnp.where(lt2, 0.0, 1.0), axis=0, keepdims=True)
    x3 = jnp.where(lt2, x2, _NEG)
    m3 = jnp.max(x3, axis=0, keepdims=True)

    b1 = jnp.minimum(c1, _K)
    b2 = jnp.minimum(_K - b1, c2)
    b3 = _K - b1 - b2
    top3 = m1 * b1 + m2 * b2 + m3 * b3

    return jnp.sum(top3 * rq)


def _knn_step(q_ref, s_ref, out_ref):
    q = q_ref[0]
    rq = jax.lax.rsqrt(jnp.maximum(jnp.sum(q * q, axis=0, keepdims=True), 1e-24))
    n_way = s_ref.shape[1]
    for c in range(n_way):
        score = _way_score(q, s_ref[0, c], rq)
        out_ref[0, c] = jnp.full((8, 128), score, dtype=jnp.float32)


def kernel(query_features, support_features):
    B, D, h, w = query_features.shape
    Way = support_features.shape[1]
    hw = h * w
    q = query_features.reshape(B, D, hw)

    scores = pl.pallas_call(
        _knn_step,
        grid=(B,),
        in_specs=[
            pl.BlockSpec((1, D, hw), lambda b: (b, 0, 0)),
            pl.BlockSpec((1, Way, D, hw), lambda b: (b, 0, 0, 0)),
        ],
        out_specs=pl.BlockSpec((1, Way, 8, 128), lambda b: (b, 0, 0, 0)),
        out_shape=jax.ShapeDtypeStruct((B, Way, 8, 128), jnp.float32),
    )(q, support_features)
    return scores[:, :, 0, 0]
